# Initial kernel scaffold; baseline (speedup 1.0000x reference)
import jax
import jax.numpy as jnp
from jax.experimental import pallas as pl


def kernel(v, edge_index, r_ij, S, d, dW, dV, params):
    raise NotImplementedError("write your pallas kernel here")


# speedup vs baseline: 6.1139x; 6.1139x over previous
import functools

import jax
import jax.numpy as jnp
from jax import lax
from jax.experimental import pallas as pl

H_SMOOTH = 2.0
EPS = 0.001
EPS2 = 0.01
EPS3 = 0.001
BE = 6400

_HI = jax.lax.Precision.HIGHEST


def _dotg(a, b):
    return lax.dot_general(a, b, (((1,), (0,)), ((), ())), precision=_HI,
                           preferred_element_type=jnp.float32)


def _edge_body(rT, dVT, dWT, fiT, fjT, P1, B12, W2s, W3s, SCL,
               pi_ref, pj_ref):
    rij = rT[...]
    r = jnp.sqrt(jnp.sum(rij * rij, axis=0, keepdims=True))
    e = rij / (r + 1e-8)
    s = r / H_SMOOTH

    vi = fiT[0:3, :]
    vj = fjT[0:3, :]
    Ti = fiT[3:4, :]
    Tj = fjT[3:4, :]
    invTi = fiT[4:5, :]
    invTj = fjT[4:5, :]
    p2i = fiT[5:6, :]
    p2j = fjT[5:6, :]
    invCi = fiT[6:7, :]
    invCj = fjT[6:7, :]
    invCTi = fiT[7:8, :]
    invCTj = fjT[7:8, :]

    vij = vi - vj
    ev = jnp.sum(e * vij, axis=0, keepdims=True)
    vv = jnp.sum(vij * vij, axis=0, keepdims=True)

    kB = SCL[0:1, 4:5]
    invm = SCL[0:1, 5:6]
    s2kB_invm = SCL[0:1, 6:7]

    def mlp_abc(c, Trow):
        ws = P1[:, 2 * c:2 * c + 1]
        wt = P1[:, 2 * c + 1:2 * c + 2]
        b1 = B12[:, c:c + 1]
        b2 = B12[:, 4 + c:4 + c + 1]
        h1 = jax.nn.silu(ws * s + wt * Trow + b1)
        h2 = jax.nn.silu(_dotg(W2s[:, 32 * c:32 * c + 32], h1) + b2)
        return _dotg(W3s[c:c + 1, :], h2) + SCL[0:1, c:c + 1]

    def mlp_w(sp):
        h1 = jax.nn.silu(P1[:, 6:7] * sp + B12[:, 3:4])
        h2 = jax.nn.silu(_dotg(W2s[:, 96:128], h1) + B12[:, 7:8])
        out = _dotg(W3s[3:4, :], h2) + SCL[0:1, 3:4]
        return jnp.exp(out) * (1.0 - sp * sp)

    sp_p = (r + EPS) / H_SMOOTH
    sp_m = (r - EPS) / H_SMOOTH
    dW_dr = (mlp_w(sp_p) - mlp_w(sp_m)) / (2 * EPS)
    gW = dW_dr * e

    Ai = mlp_abc(0, Ti)
    Aj = mlp_abc(0, Tj)
    Aie = mlp_abc(0, Ti + EPS3)
    Aje = mlp_abc(0, Tj + EPS3)
    Bi = mlp_abc(1, Ti)
    Bj = mlp_abc(1, Tj)
    Bie = mlp_abc(1, Ti + EPS3)
    Bje = mlp_abc(1, Tj + EPS3)
    Ci = mlp_abc(2, Ti)
    Cj = mlp_abc(2, Tj)
    Cie = mlp_abc(2, Ti + EPS3)
    Cje = mlp_abc(2, Tj + EPS3)

    Aij = Ai * Aj
    Bij = Bi * Bj
    Cij = Ci * Cj
    gA_i = 2.0 * Aij * (Aie * Aj - Aij) / EPS3
    gB_i = 2.0 * Bij * (Bie * Bj - Bij) / EPS3
    gC_i = 2.0 * Cij * (Cie * Cj - Cij) / EPS3
    gA_j = 2.0 * Aij * (Ai * Aje - Aij) / EPS3
    gB_j = 2.0 * Bij * (Bi * Bje - Bij) / EPS3
    gC_j = 2.0 * Cij * (Ci * Cje - Cij) / EPS3

    A2 = Aij * Aij
    B2 = Bij * Bij
    C2 = Cij * Cij

    termPd = (p2i + p2j) * gW
    aux = A2 / 2 * vij + (A2 / 2 + (B2 - A2) / 3.0) * ev * e
    term = (invTi + invTj) * aux
    term1 = -(invCTi + invCTj) * aux
    term2 = ((gA_i / 2 * vij + (gA_i / 2 + (gB_i - gA_i) / 3.0) * ev * e)
             * invCi
             + (gA_j / 2 * vij + (gA_j / 2 + (gB_j - gA_j) / 3.0) * ev * e)
             * invCj)
    msg_v = termPd + 0.5 * term + 0.5 * kB * (term1 + term2)
    pvec_i = -invm * msg_v
    pvec_j = invm * msg_v

    aux2 = (A2 / 2 * vv + (A2 / 2 + (B2 - A2) / 3.0) * ev * ev) / 4
    t1 = -(2 * invCTi + invCTj) * aux2
    t2 = ((gA_i / 2 * vv + (gA_i / 2 + (gB_i - gA_i) / 3.0) * ev * ev)
          * invCi / 4
          + (gA_j / 2 * vv + (gA_j / 2 + (gB_j - gA_j) / 3.0) * ev * ev)
          * invCj / 4)
    t4 = -(2 * invCTi - invCTj) * C2
    t5 = gC_i * invCi - gC_j * invCj
    t6 = -(4.0 * A2 / 2 + (B2 - A2) / 3.0)
    common = (invTi + invTj) * aux2 + kB * (t1 + t2 + t6 * invm)
    anti = (invTi - invTj) * C2 + kB * (t4 + t5)
    ps_i = common + anti
    ps_j = common - anti

    dw = dWT[...]
    tr3 = (dw[0:1] + dw[4:5] + dw[8:9]) / 3.0
    q01 = (dw[1:2] + dw[3:4]) * 0.5
    q02 = (dw[2:3] + dw[6:7]) * 0.5
    q12 = (dw[5:6] + dw[7:8]) * 0.5
    e0 = e[0:1]
    e1 = e[1:2]
    e2 = e[2:3]
    sde0 = dw[0:1] * e0 + q01 * e1 + q02 * e2
    sde1 = q01 * e0 + dw[4:5] * e1 + q12 * e2
    sde2 = q02 * e0 + q12 * e1 + dw[8:9] * e2
    symdote = jnp.concatenate([sde0, sde1, sde2], axis=0)
    wterm = Aij * symdote + (Bij - Aij) * tr3 * e
    pw_i = s2kB_invm * wterm
    pw_j = -pw_i

    st = -0.5 * jnp.sum(wterm * vij, axis=0, keepdims=True)
    sc = Cij * dVT[...]
    pt_i = st + sc
    pt_j = st - sc

    pi_ref[...] = jnp.concatenate([pvec_i, ps_i, pw_i, pt_i], axis=0)
    pj_ref[...] = jnp.concatenate([pvec_j, ps_j, pw_j, pt_j], axis=0)


def _mono_ref(ps, x):
    t = jnp.array([1.0, -1.0], dtype=x.dtype)
    n = len(ps)
    for idx in range(n):
        W, b = ps[idx]
        Wm = jnp.abs(W) * t if idx == 0 else jnp.abs(W)
        x = x @ Wm.T + b
        if idx < n - 1:
            x = jax.nn.softplus(x)
    return x


def kernel(v, edge_index, r_ij, S, d, dW, dV, params):
    N = v.shape[0]
    E = edge_index.shape[1]
    i = edge_index[0]
    j = edge_index[1]
    kB = jnp.exp(params['log_k_B'])
    m = jnp.exp(params['log_m'])
    invm = 1.0 / m
    s2kB = jnp.sqrt(2 * kB)

    V = 1.0 / d
    S_perturb = jnp.concatenate([S, S + EPS2, S, S - EPS2], axis=0)
    V_perturb = jnp.concatenate([V, V, V + EPS2, V], axis=0)
    U_cat = _mono_ref(params['U'],
                      jnp.concatenate([S_perturb, V_perturb], axis=-1))
    U = U_cat[0:N]
    U_Splus = U_cat[N:2 * N]
    U_Vplus = U_cat[2 * N:3 * N]
    U_Sminus = U_cat[3 * N:4 * N]
    T = (U_Splus - U) / EPS2
    P = -(U_Vplus - U) / EPS2
    C = T * EPS2 ** 2 / (U_Splus - 2 * U + U_Sminus)
    invT = 1.0 / T
    p2 = P / d ** 2
    invC = 1.0 / C
    invCT = 1.0 / C / T

    Ft = jnp.concatenate([v, T, invT, p2, invC, invCT], axis=1).T
    fiT = jnp.take(Ft, i, axis=1)
    fjT = jnp.take(Ft, j, axis=1)

    rT = r_ij.T
    dWT = dW.reshape(E, 9).T
    dVT = dV.T

    (W1A, b1A), (W2A, b2A), (W3A, b3A) = params['A']
    (W1B, b1B), (W2B, b2B), (W3B, b3B) = params['B']
    (W1C, b1C), (W2C, b2C), (W3C, b3C) = params['C']
    (W1W, b1W), (W2W, b2W), (W3W, b3W) = params['W']
    z32 = jnp.zeros((32,), jnp.float32)
    P1 = jnp.stack([W1A[:, 0], W1A[:, 1], W1B[:, 0], W1B[:, 1],
                    W1C[:, 0], W1C[:, 1], W1W[:, 0], z32], axis=1)
    B12 = jnp.stack([b1A, b1B, b1C, b1W, b2A, b2B, b2C, b2W], axis=1)
    W2s = jnp.concatenate([W2A, W2B, W2C, W2W], axis=1)
    W3s = jnp.concatenate([W3A, W3B, W3C, W3W], axis=0)
    SCL = jnp.stack([b3A[0], b3B[0], b3C[0], b3W[0],
                     kB, invm, s2kB * invm,
                     jnp.float32(0.0)]).reshape(1, 8)

    nblk = E // BE
    full = lambda n: (0, n)
    fix = lambda n: (0, 0)
    piT, pjT = pl.pallas_call(
        _edge_body,
        grid=(nblk,),
        in_specs=[
            pl.BlockSpec((3, BE), full),
            pl.BlockSpec((1, BE), full),
            pl.BlockSpec((9, BE), full),
            pl.BlockSpec((8, BE), full),
            pl.BlockSpec((8, BE), full),
            pl.BlockSpec((32, 8), fix),
            pl.BlockSpec((32, 8), fix),
            pl.BlockSpec((32, 128), fix),
            pl.BlockSpec((4, 32), fix),
            pl.BlockSpec((1, 8), fix),
        ],
        out_specs=[pl.BlockSpec((8, BE), full),
                   pl.BlockSpec((8, BE), full)],
        out_shape=[jax.ShapeDtypeStruct((8, E), jnp.float32),
                   jax.ShapeDtypeStruct((8, E), jnp.float32)],
    )(rT, dVT, dWT, fiT, fjT, P1, B12, W2s, W3s, SCL)

    acc = jnp.zeros((N, 8), jnp.float32).at[i].add(piT.T).at[j].add(pjT.T)
    out = jnp.concatenate([acc[:, 0:3],
                           acc[:, 3:4] * invT,
                           acc[:, 4:7],
                           acc[:, 7:8] * (s2kB * invT)], axis=1)
    return out


# speedup vs baseline: 8.9576x; 1.4651x over previous
import functools

import jax
import jax.numpy as jnp
from jax import lax
from jax.experimental import pallas as pl
from jax.experimental.pallas import tpu as pltpu
from jax.experimental.pallas import tpu_sc as plsc

H_SMOOTH = 2.0
EPS = 0.001
EPS2 = 0.01
EPS3 = 0.001
BE = 6400

_HI = jax.lax.Precision.HIGHEST


def _dotg(a, b):
    return lax.dot_general(a, b, (((1,), (0,)), ((), ())), precision=_HI,
                           preferred_element_type=jnp.float32)


def _edge_body(rT, dVT, dWT, fiT, fjT, P1, B12, W2s, W3s, SCL,
               pi_ref, pj_ref):
    rij = rT[...]
    r = jnp.sqrt(jnp.sum(rij * rij, axis=0, keepdims=True))
    e = rij / (r + 1e-8)
    s = r / H_SMOOTH

    vi = fiT[0:3, :]
    vj = fjT[0:3, :]
    Ti = fiT[3:4, :]
    Tj = fjT[3:4, :]
    invTi = fiT[4:5, :]
    invTj = fjT[4:5, :]
    p2i = fiT[5:6, :]
    p2j = fjT[5:6, :]
    invCi = fiT[6:7, :]
    invCj = fjT[6:7, :]
    invCTi = fiT[7:8, :]
    invCTj = fjT[7:8, :]

    vij = vi - vj
    ev = jnp.sum(e * vij, axis=0, keepdims=True)
    vv = jnp.sum(vij * vij, axis=0, keepdims=True)

    kB = SCL[0:1, 4:5]
    invm = SCL[0:1, 5:6]
    s2kB_invm = SCL[0:1, 6:7]

    def mlp_abc(c, Trow):
        ws = P1[:, 2 * c:2 * c + 1]
        wt = P1[:, 2 * c + 1:2 * c + 2]
        b1 = B12[:, c:c + 1]
        b2 = B12[:, 4 + c:4 + c + 1]
        h1 = jax.nn.silu(ws * s + wt * Trow + b1)
        h2 = jax.nn.silu(_dotg(W2s[:, 32 * c:32 * c + 32], h1) + b2)
        return _dotg(W3s[c:c + 1, :], h2) + SCL[0:1, c:c + 1]

    def mlp_w(sp):
        h1 = jax.nn.silu(P1[:, 6:7] * sp + B12[:, 3:4])
        h2 = jax.nn.silu(_dotg(W2s[:, 96:128], h1) + B12[:, 7:8])
        out = _dotg(W3s[3:4, :], h2) + SCL[0:1, 3:4]
        return jnp.exp(out) * (1.0 - sp * sp)

    sp_p = (r + EPS) / H_SMOOTH
    sp_m = (r - EPS) / H_SMOOTH
    dW_dr = (mlp_w(sp_p) - mlp_w(sp_m)) / (2 * EPS)
    gW = dW_dr * e

    Ai = mlp_abc(0, Ti)
    Aj = mlp_abc(0, Tj)
    Aie = mlp_abc(0, Ti + EPS3)
    Aje = mlp_abc(0, Tj + EPS3)
    Bi = mlp_abc(1, Ti)
    Bj = mlp_abc(1, Tj)
    Bie = mlp_abc(1, Ti + EPS3)
    Bje = mlp_abc(1, Tj + EPS3)
    Ci = mlp_abc(2, Ti)
    Cj = mlp_abc(2, Tj)
    Cie = mlp_abc(2, Ti + EPS3)
    Cje = mlp_abc(2, Tj + EPS3)

    Aij = Ai * Aj
    Bij = Bi * Bj
    Cij = Ci * Cj
    gA_i = 2.0 * Aij * (Aie * Aj - Aij) / EPS3
    gB_i = 2.0 * Bij * (Bie * Bj - Bij) / EPS3
    gC_i = 2.0 * Cij * (Cie * Cj - Cij) / EPS3
    gA_j = 2.0 * Aij * (Ai * Aje - Aij) / EPS3
    gB_j = 2.0 * Bij * (Bi * Bje - Bij) / EPS3
    gC_j = 2.0 * Cij * (Ci * Cje - Cij) / EPS3

    A2 = Aij * Aij
    B2 = Bij * Bij
    C2 = Cij * Cij

    termPd = (p2i + p2j) * gW
    aux = A2 / 2 * vij + (A2 / 2 + (B2 - A2) / 3.0) * ev * e
    term = (invTi + invTj) * aux
    term1 = -(invCTi + invCTj) * aux
    term2 = ((gA_i / 2 * vij + (gA_i / 2 + (gB_i - gA_i) / 3.0) * ev * e)
             * invCi
             + (gA_j / 2 * vij + (gA_j / 2 + (gB_j - gA_j) / 3.0) * ev * e)
             * invCj)
    msg_v = termPd + 0.5 * term + 0.5 * kB * (term1 + term2)
    pvec_i = -invm * msg_v
    pvec_j = invm * msg_v

    aux2 = (A2 / 2 * vv + (A2 / 2 + (B2 - A2) / 3.0) * ev * ev) / 4
    t1 = -(2 * invCTi + invCTj) * aux2
    t2 = ((gA_i / 2 * vv + (gA_i / 2 + (gB_i - gA_i) / 3.0) * ev * ev)
          * invCi / 4
          + (gA_j / 2 * vv + (gA_j / 2 + (gB_j - gA_j) / 3.0) * ev * ev)
          * invCj / 4)
    t4 = -(2 * invCTi - invCTj) * C2
    t5 = gC_i * invCi - gC_j * invCj
    t6 = -(4.0 * A2 / 2 + (B2 - A2) / 3.0)
    common = (invTi + invTj) * aux2 + kB * (t1 + t2 + t6 * invm)
    anti = (invTi - invTj) * C2 + kB * (t4 + t5)
    ps_i = common + anti
    ps_j = common - anti

    dw = dWT[...]
    tr3 = (dw[0:1] + dw[4:5] + dw[8:9]) / 3.0
    q01 = (dw[1:2] + dw[3:4]) * 0.5
    q02 = (dw[2:3] + dw[6:7]) * 0.5
    q12 = (dw[5:6] + dw[7:8]) * 0.5
    e0 = e[0:1]
    e1 = e[1:2]
    e2 = e[2:3]
    sde0 = dw[0:1] * e0 + q01 * e1 + q02 * e2
    sde1 = q01 * e0 + dw[4:5] * e1 + q12 * e2
    sde2 = q02 * e0 + q12 * e1 + dw[8:9] * e2
    symdote = jnp.concatenate([sde0, sde1, sde2], axis=0)
    wterm = Aij * symdote + (Bij - Aij) * tr3 * e
    pw_i = s2kB_invm * wterm
    pw_j = -pw_i

    st = -0.5 * jnp.sum(wterm * vij, axis=0, keepdims=True)
    sc = Cij * dVT[...]
    pt_i = st + sc
    pt_j = st - sc

    pi_ref[...] = jnp.concatenate([pvec_i, ps_i, pw_i, pt_i], axis=0)
    pj_ref[...] = jnp.concatenate([pvec_j, ps_j, pw_j, pt_j], axis=0)


_SC_NC = 2
_SC_NS = 16
_SC_NW = _SC_NC * _SC_NS


@functools.lru_cache(maxsize=None)
def _make_sc_scatter(E2, TOT, GPW, NACC):
    CH = GPW * 128
    SUBROWS = NACC // _SC_NS
    mesh = plsc.VectorSubcoreMesh(core_axis_name="c", subcore_axis_name="s")

    @functools.partial(
        pl.kernel, mesh=mesh,
        compiler_params=pltpu.CompilerParams(use_tc_tiling_on_sc=False),
        out_type=jax.ShapeDtypeStruct((_SC_NC, NACC, 8), jnp.float32),
        scratch_types=[
            pltpu.VMEM((GPW, 128), jnp.int32),
            pltpu.VMEM((CH, 8), jnp.float32),
            pltpu.VMEM_SHARED((NACC, 8), jnp.float32),
        ],
    )
    def scat(pcat_hbm, idx_hbm, zeros_hbm, out_hbm, idx_v, pay_v, acc_sh):
        c = lax.axis_index("c")
        s = lax.axis_index("s")
        wid = s * _SC_NC + c
        pltpu.sync_copy(zeros_hbm.at[pl.ds(s * SUBROWS, SUBROWS)],
                        acc_sh.at[pl.ds(s * SUBROWS, SUBROWS)])
        pltpu.sync_copy(idx_hbm.at[wid], idx_v)
        pltpu.sync_copy(pcat_hbm.at[pl.ds(wid * CH, CH)], pay_v)
        plsc.subcore_barrier()

        def body(g, _):
            pltpu.sync_copy(pay_v.at[pl.ds(g * 128, 128)],
                            acc_sh.at[idx_v.at[g]], add=True)
            return _

        lax.fori_loop(0, GPW, body, None)
        plsc.subcore_barrier()
        pltpu.sync_copy(acc_sh.at[pl.ds(s * SUBROWS, SUBROWS)],
                        out_hbm.at[c, pl.ds(s * SUBROWS, SUBROWS)])

    return scat


def _mono_ref(ps, x):
    t = jnp.array([1.0, -1.0], dtype=x.dtype)
    n = len(ps)
    for idx in range(n):
        W, b = ps[idx]
        Wm = jnp.abs(W) * t if idx == 0 else jnp.abs(W)
        x = x @ Wm.T + b
        if idx < n - 1:
            x = jax.nn.softplus(x)
    return x


def kernel(v, edge_index, r_ij, S, d, dW, dV, params):
    N = v.shape[0]
    E = edge_index.shape[1]
    i = edge_index[0]
    j = edge_index[1]
    kB = jnp.exp(params['log_k_B'])
    m = jnp.exp(params['log_m'])
    invm = 1.0 / m
    s2kB = jnp.sqrt(2 * kB)

    V = 1.0 / d
    S_perturb = jnp.concatenate([S, S + EPS2, S, S - EPS2], axis=0)
    V_perturb = jnp.concatenate([V, V, V + EPS2, V], axis=0)
    U_cat = _mono_ref(params['U'],
                      jnp.concatenate([S_perturb, V_perturb], axis=-1))
    U = U_cat[0:N]
    U_Splus = U_cat[N:2 * N]
    U_Vplus = U_cat[2 * N:3 * N]
    U_Sminus = U_cat[3 * N:4 * N]
    T = (U_Splus - U) / EPS2
    P = -(U_Vplus - U) / EPS2
    C = T * EPS2 ** 2 / (U_Splus - 2 * U + U_Sminus)
    invT = 1.0 / T
    p2 = P / d ** 2
    invC = 1.0 / C
    invCT = 1.0 / C / T

    Ft = jnp.concatenate([v, T, invT, p2, invC, invCT], axis=1).T
    fiT = jnp.take(Ft, i, axis=1)
    fjT = jnp.take(Ft, j, axis=1)

    rT = r_ij.T
    dWT = dW.reshape(E, 9).T
    dVT = dV.T

    (W1A, b1A), (W2A, b2A), (W3A, b3A) = params['A']
    (W1B, b1B), (W2B, b2B), (W3B, b3B) = params['B']
    (W1C, b1C), (W2C, b2C), (W3C, b3C) = params['C']
    (W1W, b1W), (W2W, b2W), (W3W, b3W) = params['W']
    z32 = jnp.zeros((32,), jnp.float32)
    P1 = jnp.stack([W1A[:, 0], W1A[:, 1], W1B[:, 0], W1B[:, 1],
                    W1C[:, 0], W1C[:, 1], W1W[:, 0], z32], axis=1)
    B12 = jnp.stack([b1A, b1B, b1C, b1W, b2A, b2B, b2C, b2W], axis=1)
    W2s = jnp.concatenate([W2A, W2B, W2C, W2W], axis=1)
    W3s = jnp.concatenate([W3A, W3B, W3C, W3W], axis=0)
    SCL = jnp.stack([b3A[0], b3B[0], b3C[0], b3W[0],
                     kB, invm, s2kB * invm,
                     jnp.float32(0.0)]).reshape(1, 8)

    nblk = E // BE
    full = lambda n: (0, n)
    fix = lambda n: (0, 0)
    piT, pjT = pl.pallas_call(
        _edge_body,
        grid=(nblk,),
        in_specs=[
            pl.BlockSpec((3, BE), full),
            pl.BlockSpec((1, BE), full),
            pl.BlockSpec((9, BE), full),
            pl.BlockSpec((8, BE), full),
            pl.BlockSpec((8, BE), full),
            pl.BlockSpec((32, 8), fix),
            pl.BlockSpec((32, 8), fix),
            pl.BlockSpec((32, 128), fix),
            pl.BlockSpec((4, 32), fix),
            pl.BlockSpec((1, 8), fix),
        ],
        out_specs=[pl.BlockSpec((8, BE), full),
                   pl.BlockSpec((8, BE), full)],
        out_shape=[jax.ShapeDtypeStruct((8, E), jnp.float32),
                   jax.ShapeDtypeStruct((8, E), jnp.float32)],
    )(rT, dVT, dWT, fiT, fjT, P1, B12, W2s, W3s, SCL)

    E2 = 2 * E
    GPW = -(-E2 // (_SC_NW * 128))
    TOT = _SC_NW * GPW * 128
    NACC = ((N + 1 + 127) // 128) * 128
    pcat = jnp.concatenate(
        [piT.T, pjT.T, jnp.zeros((TOT - E2, 8), jnp.float32)], axis=0)
    ij = jnp.concatenate(
        [i, j, jnp.full((TOT - E2,), N, jnp.int32)]).reshape(_SC_NW, GPW, 128)
    zeros_acc = jnp.zeros((NACC, 8), jnp.float32)
    partials = _make_sc_scatter(E2, TOT, GPW, NACC)(pcat, ij, zeros_acc)
    acc = partials[0, :N] + partials[1, :N]
    out = jnp.concatenate([acc[:, 0:3],
                           acc[:, 3:4] * invT,
                           acc[:, 4:7],
                           acc[:, 7:8] * (s2kB * invT)], axis=1)
    return out


# speedup vs baseline: 12.3809x; 1.3822x over previous
import functools

import jax
import jax.numpy as jnp
from jax import lax
from jax.experimental import pallas as pl
from jax.experimental.pallas import tpu as pltpu
from jax.experimental.pallas import tpu_sc as plsc

H_SMOOTH = 2.0
EPS = 0.001
EPS2 = 0.01
EPS3 = 0.001
BE = 6400

_HI = jax.lax.Precision.HIGHEST


def _dotg(a, b):
    return lax.dot_general(a, b, (((1,), (0,)), ((), ())), precision=_HI,
                           preferred_element_type=jnp.float32)


def _edge_body(rT, dVT, dWT, fiT, fjT, P1, B12, W2s, W3s, SCL,
               pi_ref, pj_ref):
    rij = rT[...]
    r = jnp.sqrt(jnp.sum(rij * rij, axis=0, keepdims=True))
    e = rij / (r + 1e-8)
    s = r / H_SMOOTH

    vi = fiT[0:3, :]
    vj = fjT[0:3, :]
    Ti = fiT[3:4, :]
    Tj = fjT[3:4, :]
    invTi = fiT[4:5, :]
    invTj = fjT[4:5, :]
    p2i = fiT[5:6, :]
    p2j = fjT[5:6, :]
    invCi = fiT[6:7, :]
    invCj = fjT[6:7, :]
    invCTi = fiT[7:8, :]
    invCTj = fjT[7:8, :]

    vij = vi - vj
    ev = jnp.sum(e * vij, axis=0, keepdims=True)
    vv = jnp.sum(vij * vij, axis=0, keepdims=True)

    kB = SCL[0:1, 4:5]
    invm = SCL[0:1, 5:6]
    s2kB_invm = SCL[0:1, 6:7]

    def mlp_abc(c, Trow):
        ws = P1[:, 2 * c:2 * c + 1]
        wt = P1[:, 2 * c + 1:2 * c + 2]
        b1 = B12[:, c:c + 1]
        b2 = B12[:, 4 + c:4 + c + 1]
        h1 = jax.nn.silu(ws * s + wt * Trow + b1)
        h2 = jax.nn.silu(_dotg(W2s[:, 32 * c:32 * c + 32], h1) + b2)
        return _dotg(W3s[c:c + 1, :], h2) + SCL[0:1, c:c + 1]

    def mlp_w(sp):
        h1 = jax.nn.silu(P1[:, 6:7] * sp + B12[:, 3:4])
        h2 = jax.nn.silu(_dotg(W2s[:, 96:128], h1) + B12[:, 7:8])
        out = _dotg(W3s[3:4, :], h2) + SCL[0:1, 3:4]
        return jnp.exp(out) * (1.0 - sp * sp)

    sp_p = (r + EPS) / H_SMOOTH
    sp_m = (r - EPS) / H_SMOOTH
    dW_dr = (mlp_w(sp_p) - mlp_w(sp_m)) / (2 * EPS)
    gW = dW_dr * e

    Ai = mlp_abc(0, Ti)
    Aj = mlp_abc(0, Tj)
    Aie = mlp_abc(0, Ti + EPS3)
    Aje = mlp_abc(0, Tj + EPS3)
    Bi = mlp_abc(1, Ti)
    Bj = mlp_abc(1, Tj)
    Bie = mlp_abc(1, Ti + EPS3)
    Bje = mlp_abc(1, Tj + EPS3)
    Ci = mlp_abc(2, Ti)
    Cj = mlp_abc(2, Tj)
    Cie = mlp_abc(2, Ti + EPS3)
    Cje = mlp_abc(2, Tj + EPS3)

    Aij = Ai * Aj
    Bij = Bi * Bj
    Cij = Ci * Cj
    gA_i = 2.0 * Aij * (Aie * Aj - Aij) / EPS3
    gB_i = 2.0 * Bij * (Bie * Bj - Bij) / EPS3
    gC_i = 2.0 * Cij * (Cie * Cj - Cij) / EPS3
    gA_j = 2.0 * Aij * (Ai * Aje - Aij) / EPS3
    gB_j = 2.0 * Bij * (Bi * Bje - Bij) / EPS3
    gC_j = 2.0 * Cij * (Ci * Cje - Cij) / EPS3

    A2 = Aij * Aij
    B2 = Bij * Bij
    C2 = Cij * Cij

    termPd = (p2i + p2j) * gW
    aux = A2 / 2 * vij + (A2 / 2 + (B2 - A2) / 3.0) * ev * e
    term = (invTi + invTj) * aux
    term1 = -(invCTi + invCTj) * aux
    term2 = ((gA_i / 2 * vij + (gA_i / 2 + (gB_i - gA_i) / 3.0) * ev * e)
             * invCi
             + (gA_j / 2 * vij + (gA_j / 2 + (gB_j - gA_j) / 3.0) * ev * e)
             * invCj)
    msg_v = termPd + 0.5 * term + 0.5 * kB * (term1 + term2)
    pvec_i = -invm * msg_v
    pvec_j = invm * msg_v

    aux2 = (A2 / 2 * vv + (A2 / 2 + (B2 - A2) / 3.0) * ev * ev) / 4
    t1 = -(2 * invCTi + invCTj) * aux2
    t2 = ((gA_i / 2 * vv + (gA_i / 2 + (gB_i - gA_i) / 3.0) * ev * ev)
          * invCi / 4
          + (gA_j / 2 * vv + (gA_j / 2 + (gB_j - gA_j) / 3.0) * ev * ev)
          * invCj / 4)
    t4 = -(2 * invCTi - invCTj) * C2
    t5 = gC_i * invCi - gC_j * invCj
    t6 = -(4.0 * A2 / 2 + (B2 - A2) / 3.0)
    common = (invTi + invTj) * aux2 + kB * (t1 + t2 + t6 * invm)
    anti = (invTi - invTj) * C2 + kB * (t4 + t5)
    ps_i = common + anti
    ps_j = common - anti

    dw = dWT[...]
    tr3 = (dw[0:1] + dw[4:5] + dw[8:9]) / 3.0
    q01 = (dw[1:2] + dw[3:4]) * 0.5
    q02 = (dw[2:3] + dw[6:7]) * 0.5
    q12 = (dw[5:6] + dw[7:8]) * 0.5
    e0 = e[0:1]
    e1 = e[1:2]
    e2 = e[2:3]
    sde0 = dw[0:1] * e0 + q01 * e1 + q02 * e2
    sde1 = q01 * e0 + dw[4:5] * e1 + q12 * e2
    sde2 = q02 * e0 + q12 * e1 + dw[8:9] * e2
    symdote = jnp.concatenate([sde0, sde1, sde2], axis=0)
    wterm = Aij * symdote + (Bij - Aij) * tr3 * e
    pw_i = s2kB_invm * wterm
    pw_j = -pw_i

    st = -0.5 * jnp.sum(wterm * vij, axis=0, keepdims=True)
    sc = Cij * dVT[...]
    pt_i = st + sc
    pt_j = st - sc

    pi_ref[...] = jnp.concatenate([pvec_i, ps_i, pw_i, pt_i], axis=0)
    pj_ref[...] = jnp.concatenate([pvec_j, ps_j, pw_j, pt_j], axis=0)


_SC_NC = 2
_SC_NS = 16
_SC_NW = _SC_NC * _SC_NS


@functools.lru_cache(maxsize=None)
def _make_sc_scatter(E2, TOT, GPW, NACC):
    CH = GPW * 128
    SUBROWS = NACC // _SC_NS
    mesh = plsc.VectorSubcoreMesh(core_axis_name="c", subcore_axis_name="s")

    @functools.partial(
        pl.kernel, mesh=mesh,
        compiler_params=pltpu.CompilerParams(use_tc_tiling_on_sc=False),
        out_type=jax.ShapeDtypeStruct((_SC_NC, NACC, 8), jnp.float32),
        scratch_types=[
            pltpu.VMEM((GPW, 128), jnp.int32),
            pltpu.VMEM((CH, 8), jnp.float32),
            pltpu.VMEM_SHARED((NACC, 8), jnp.float32),
        ],
    )
    def scat(pcat_hbm, idx_hbm, zeros_hbm, out_hbm, idx_v, pay_v, acc_sh):
        c = lax.axis_index("c")
        s = lax.axis_index("s")
        wid = s * _SC_NC + c
        pltpu.sync_copy(zeros_hbm.at[pl.ds(s * SUBROWS, SUBROWS)],
                        acc_sh.at[pl.ds(s * SUBROWS, SUBROWS)])
        pltpu.sync_copy(idx_hbm.at[wid], idx_v)
        pltpu.sync_copy(pcat_hbm.at[pl.ds(wid * CH, CH)], pay_v)
        plsc.subcore_barrier()

        def body(g, _):
            pltpu.sync_copy(pay_v.at[pl.ds(g * 128, 128)],
                            acc_sh.at[idx_v.at[g]], add=True)
            return _

        lax.fori_loop(0, GPW, body, None)
        plsc.subcore_barrier()
        pltpu.sync_copy(acc_sh.at[pl.ds(s * SUBROWS, SUBROWS)],
                        out_hbm.at[c, pl.ds(s * SUBROWS, SUBROWS)])

    return scat


@functools.lru_cache(maxsize=None)
def _make_sc_gather(TOT, GPW):
    CH = GPW * 128

    @functools.partial(
        pl.kernel,
        mesh=plsc.VectorSubcoreMesh(core_axis_name="c", subcore_axis_name="s"),
        compiler_params=pltpu.CompilerParams(use_tc_tiling_on_sc=False),
        out_type=jax.ShapeDtypeStruct((TOT, 8), jnp.float32),
        scratch_types=[
            pltpu.VMEM((GPW, 128), jnp.int32),
            pltpu.VMEM((CH, 8), jnp.float32),
        ],
    )
    def gath(tab_hbm, idx_hbm, out_hbm, idx_v, rows_v):
        c = lax.axis_index("c")
        s = lax.axis_index("s")
        wid = s * _SC_NC + c
        pltpu.sync_copy(idx_hbm.at[wid], idx_v)

        def body(g, _):
            pltpu.sync_copy(tab_hbm.at[idx_v.at[g]],
                            rows_v.at[pl.ds(g * 128, 128)])
            return _

        lax.fori_loop(0, GPW, body, None)
        pltpu.sync_copy(rows_v, out_hbm.at[pl.ds(wid * CH, CH)])

    return gath


def _mono_ref(ps, x):
    t = jnp.array([1.0, -1.0], dtype=x.dtype)
    n = len(ps)
    for idx in range(n):
        W, b = ps[idx]
        Wm = jnp.abs(W) * t if idx == 0 else jnp.abs(W)
        x = x @ Wm.T + b
        if idx < n - 1:
            x = jax.nn.softplus(x)
    return x


def kernel(v, edge_index, r_ij, S, d, dW, dV, params):
    N = v.shape[0]
    E = edge_index.shape[1]
    i = edge_index[0]
    j = edge_index[1]
    kB = jnp.exp(params['log_k_B'])
    m = jnp.exp(params['log_m'])
    invm = 1.0 / m
    s2kB = jnp.sqrt(2 * kB)

    V = 1.0 / d
    S_perturb = jnp.concatenate([S, S + EPS2, S, S - EPS2], axis=0)
    V_perturb = jnp.concatenate([V, V, V + EPS2, V], axis=0)
    U_cat = _mono_ref(params['U'],
                      jnp.concatenate([S_perturb, V_perturb], axis=-1))
    U = U_cat[0:N]
    U_Splus = U_cat[N:2 * N]
    U_Vplus = U_cat[2 * N:3 * N]
    U_Sminus = U_cat[3 * N:4 * N]
    T = (U_Splus - U) / EPS2
    P = -(U_Vplus - U) / EPS2
    C = T * EPS2 ** 2 / (U_Splus - 2 * U + U_Sminus)
    invT = 1.0 / T
    p2 = P / d ** 2
    invC = 1.0 / C
    invCT = 1.0 / C / T

    F = jnp.concatenate([v, T, invT, p2, invC, invCT], axis=1)
    E2 = 2 * E
    GPW = -(-E2 // (_SC_NW * 128))
    TOT = _SC_NW * GPW * 128
    ij_g = jnp.concatenate(
        [i, j, jnp.zeros((TOT - E2,), jnp.int32)]).reshape(_SC_NW, GPW, 128)
    g_rows = _make_sc_gather(TOT, GPW)(F, ij_g)
    fiT = g_rows[:E].T
    fjT = g_rows[E:E2].T

    rT = r_ij.T
    dWT = dW.reshape(E, 9).T
    dVT = dV.T

    (W1A, b1A), (W2A, b2A), (W3A, b3A) = params['A']
    (W1B, b1B), (W2B, b2B), (W3B, b3B) = params['B']
    (W1C, b1C), (W2C, b2C), (W3C, b3C) = params['C']
    (W1W, b1W), (W2W, b2W), (W3W, b3W) = params['W']
    z32 = jnp.zeros((32,), jnp.float32)
    P1 = jnp.stack([W1A[:, 0], W1A[:, 1], W1B[:, 0], W1B[:, 1],
                    W1C[:, 0], W1C[:, 1], W1W[:, 0], z32], axis=1)
    B12 = jnp.stack([b1A, b1B, b1C, b1W, b2A, b2B, b2C, b2W], axis=1)
    W2s = jnp.concatenate([W2A, W2B, W2C, W2W], axis=1)
    W3s = jnp.concatenate([W3A, W3B, W3C, W3W], axis=0)
    SCL = jnp.stack([b3A[0], b3B[0], b3C[0], b3W[0],
                     kB, invm, s2kB * invm,
                     jnp.float32(0.0)]).reshape(1, 8)

    nblk = E // BE
    full = lambda n: (0, n)
    fix = lambda n: (0, 0)
    piT, pjT = pl.pallas_call(
        _edge_body,
        grid=(nblk,),
        in_specs=[
            pl.BlockSpec((3, BE), full),
            pl.BlockSpec((1, BE), full),
            pl.BlockSpec((9, BE), full),
            pl.BlockSpec((8, BE), full),
            pl.BlockSpec((8, BE), full),
            pl.BlockSpec((32, 8), fix),
            pl.BlockSpec((32, 8), fix),
            pl.BlockSpec((32, 128), fix),
            pl.BlockSpec((4, 32), fix),
            pl.BlockSpec((1, 8), fix),
        ],
        out_specs=[pl.BlockSpec((8, BE), full),
                   pl.BlockSpec((8, BE), full)],
        out_shape=[jax.ShapeDtypeStruct((8, E), jnp.float32),
                   jax.ShapeDtypeStruct((8, E), jnp.float32)],
    )(rT, dVT, dWT, fiT, fjT, P1, B12, W2s, W3s, SCL)

    NACC = ((N + 1 + 127) // 128) * 128
    pcat = jnp.concatenate(
        [piT.T, pjT.T, jnp.zeros((TOT - E2, 8), jnp.float32)], axis=0)
    ij = jnp.concatenate(
        [i, j, jnp.full((TOT - E2,), N, jnp.int32)]).reshape(_SC_NW, GPW, 128)
    zeros_acc = jnp.zeros((NACC, 8), jnp.float32)
    partials = _make_sc_scatter(E2, TOT, GPW, NACC)(pcat, ij, zeros_acc)
    acc = partials[0, :N] + partials[1, :N]
    out = jnp.concatenate([acc[:, 0:3],
                           acc[:, 3:4] * invT,
                           acc[:, 4:7],
                           acc[:, 7:8] * (s2kB * invT)], axis=1)
    return out


# speedup vs baseline: 14.6860x; 1.1862x over previous
import functools

import jax
import jax.numpy as jnp
from jax import lax
from jax.experimental import pallas as pl
from jax.experimental.pallas import tpu as pltpu
from jax.experimental.pallas import tpu_sc as plsc

H_SMOOTH = 2.0
EPS = 0.001
EPS2 = 0.01
EPS3 = 0.001
BE = 6400

_HI = jax.lax.Precision.HIGHEST


def _dotg(a, b):
    return lax.dot_general(a, b, (((1,), (0,)), ((), ())), precision=_HI,
                           preferred_element_type=jnp.float32)


def _edge_body(rT, dVT, dWT, fi_blk, fj_blk, P1, B12, W2s, W3s, SCL,
               pi_ref, pj_ref):
    fiT = jnp.transpose(fi_blk[...], (1, 0))
    fjT = jnp.transpose(fj_blk[...], (1, 0))
    rij = rT[...]
    r = jnp.sqrt(jnp.sum(rij * rij, axis=0, keepdims=True))
    e = rij / (r + 1e-8)
    s = r / H_SMOOTH

    vi = fiT[0:3, :]
    vj = fjT[0:3, :]
    Ti = fiT[3:4, :]
    Tj = fjT[3:4, :]
    invTi = fiT[4:5, :]
    invTj = fjT[4:5, :]
    p2i = fiT[5:6, :]
    p2j = fjT[5:6, :]
    invCi = fiT[6:7, :]
    invCj = fjT[6:7, :]
    invCTi = fiT[7:8, :]
    invCTj = fjT[7:8, :]

    vij = vi - vj
    ev = jnp.sum(e * vij, axis=0, keepdims=True)
    vv = jnp.sum(vij * vij, axis=0, keepdims=True)

    kB = SCL[0:1, 4:5]
    invm = SCL[0:1, 5:6]
    s2kB_invm = SCL[0:1, 6:7]

    def mlp_abc(c, Trow):
        ws = P1[:, 2 * c:2 * c + 1]
        wt = P1[:, 2 * c + 1:2 * c + 2]
        b1 = B12[:, c:c + 1]
        b2 = B12[:, 4 + c:4 + c + 1]
        h1 = jax.nn.silu(ws * s + wt * Trow + b1)
        h2 = jax.nn.silu(_dotg(W2s[:, 32 * c:32 * c + 32], h1) + b2)
        return _dotg(W3s[c:c + 1, :], h2) + SCL[0:1, c:c + 1]

    def mlp_w(sp):
        h1 = jax.nn.silu(P1[:, 6:7] * sp + B12[:, 3:4])
        h2 = jax.nn.silu(_dotg(W2s[:, 96:128], h1) + B12[:, 7:8])
        out = _dotg(W3s[3:4, :], h2) + SCL[0:1, 3:4]
        return jnp.exp(out) * (1.0 - sp * sp)

    sp_p = (r + EPS) / H_SMOOTH
    sp_m = (r - EPS) / H_SMOOTH
    dW_dr = (mlp_w(sp_p) - mlp_w(sp_m)) / (2 * EPS)
    gW = dW_dr * e

    Ai = mlp_abc(0, Ti)
    Aj = mlp_abc(0, Tj)
    Aie = mlp_abc(0, Ti + EPS3)
    Aje = mlp_abc(0, Tj + EPS3)
    Bi = mlp_abc(1, Ti)
    Bj = mlp_abc(1, Tj)
    Bie = mlp_abc(1, Ti + EPS3)
    Bje = mlp_abc(1, Tj + EPS3)
    Ci = mlp_abc(2, Ti)
    Cj = mlp_abc(2, Tj)
    Cie = mlp_abc(2, Ti + EPS3)
    Cje = mlp_abc(2, Tj + EPS3)

    Aij = Ai * Aj
    Bij = Bi * Bj
    Cij = Ci * Cj
    gA_i = 2.0 * Aij * (Aie * Aj - Aij) / EPS3
    gB_i = 2.0 * Bij * (Bie * Bj - Bij) / EPS3
    gC_i = 2.0 * Cij * (Cie * Cj - Cij) / EPS3
    gA_j = 2.0 * Aij * (Ai * Aje - Aij) / EPS3
    gB_j = 2.0 * Bij * (Bi * Bje - Bij) / EPS3
    gC_j = 2.0 * Cij * (Ci * Cje - Cij) / EPS3

    A2 = Aij * Aij
    B2 = Bij * Bij
    C2 = Cij * Cij

    termPd = (p2i + p2j) * gW
    aux = A2 / 2 * vij + (A2 / 2 + (B2 - A2) / 3.0) * ev * e
    term = (invTi + invTj) * aux
    term1 = -(invCTi + invCTj) * aux
    term2 = ((gA_i / 2 * vij + (gA_i / 2 + (gB_i - gA_i) / 3.0) * ev * e)
             * invCi
             + (gA_j / 2 * vij + (gA_j / 2 + (gB_j - gA_j) / 3.0) * ev * e)
             * invCj)
    msg_v = termPd + 0.5 * term + 0.5 * kB * (term1 + term2)
    pvec_i = -invm * msg_v
    pvec_j = invm * msg_v

    aux2 = (A2 / 2 * vv + (A2 / 2 + (B2 - A2) / 3.0) * ev * ev) / 4
    t1 = -(2 * invCTi + invCTj) * aux2
    t2 = ((gA_i / 2 * vv + (gA_i / 2 + (gB_i - gA_i) / 3.0) * ev * ev)
          * invCi / 4
          + (gA_j / 2 * vv + (gA_j / 2 + (gB_j - gA_j) / 3.0) * ev * ev)
          * invCj / 4)
    t4 = -(2 * invCTi - invCTj) * C2
    t5 = gC_i * invCi - gC_j * invCj
    t6 = -(4.0 * A2 / 2 + (B2 - A2) / 3.0)
    common = (invTi + invTj) * aux2 + kB * (t1 + t2 + t6 * invm)
    anti = (invTi - invTj) * C2 + kB * (t4 + t5)
    ps_i = common + anti
    ps_j = common - anti

    dw = dWT[...]
    tr3 = (dw[0:1] + dw[4:5] + dw[8:9]) / 3.0
    q01 = (dw[1:2] + dw[3:4]) * 0.5
    q02 = (dw[2:3] + dw[6:7]) * 0.5
    q12 = (dw[5:6] + dw[7:8]) * 0.5
    e0 = e[0:1]
    e1 = e[1:2]
    e2 = e[2:3]
    sde0 = dw[0:1] * e0 + q01 * e1 + q02 * e2
    sde1 = q01 * e0 + dw[4:5] * e1 + q12 * e2
    sde2 = q02 * e0 + q12 * e1 + dw[8:9] * e2
    symdote = jnp.concatenate([sde0, sde1, sde2], axis=0)
    wterm = Aij * symdote + (Bij - Aij) * tr3 * e
    pw_i = s2kB_invm * wterm
    pw_j = -pw_i

    st = -0.5 * jnp.sum(wterm * vij, axis=0, keepdims=True)
    sc = Cij * dVT[...]
    pt_i = st + sc
    pt_j = st - sc

    pi_ref[...] = jnp.transpose(
        jnp.concatenate([pvec_i, ps_i, pw_i, pt_i], axis=0), (1, 0))
    pj_ref[...] = jnp.transpose(
        jnp.concatenate([pvec_j, ps_j, pw_j, pt_j], axis=0), (1, 0))


_SC_NC = 2
_SC_NS = 16
_SC_NW = _SC_NC * _SC_NS


@functools.lru_cache(maxsize=None)
def _make_sc_scatter(EP, GW, NACC):
    CH = GW * 128
    SUBROWS = NACC // _SC_NS
    mesh = plsc.VectorSubcoreMesh(core_axis_name="c", subcore_axis_name="s")

    @functools.partial(
        pl.kernel, mesh=mesh,
        compiler_params=pltpu.CompilerParams(use_tc_tiling_on_sc=False),
        out_type=jax.ShapeDtypeStruct((_SC_NC, NACC, 8), jnp.float32),
        scratch_types=[
            pltpu.VMEM((GW, 128), jnp.int32),
            pltpu.VMEM((CH, 8), jnp.float32),
            pltpu.VMEM_SHARED((NACC, 8), jnp.float32),
        ],
    )
    def scat(pi_hbm, pj_hbm, idx_hbm, zeros_hbm, out_hbm,
             idx_v, pay_v, acc_sh):
        c = lax.axis_index("c")
        s = lax.axis_index("s")
        wid = s * _SC_NC + c
        pltpu.sync_copy(zeros_hbm.at[pl.ds(s * SUBROWS, SUBROWS)],
                        acc_sh.at[pl.ds(s * SUBROWS, SUBROWS)])
        plsc.subcore_barrier()

        def one_stream(t, pay_hbm):
            pltpu.sync_copy(idx_hbm.at[t, pl.ds(wid * GW, GW)], idx_v)
            pltpu.sync_copy(pay_hbm.at[pl.ds(wid * CH, CH)], pay_v)

            def body(g, _):
                pltpu.sync_copy(pay_v.at[pl.ds(g * 128, 128)],
                                acc_sh.at[idx_v.at[g]], add=True)
                return _

            lax.fori_loop(0, GW, body, None)

        one_stream(0, pi_hbm)
        one_stream(1, pj_hbm)
        plsc.subcore_barrier()
        pltpu.sync_copy(acc_sh.at[pl.ds(s * SUBROWS, SUBROWS)],
                        out_hbm.at[c, pl.ds(s * SUBROWS, SUBROWS)])

    return scat


@functools.lru_cache(maxsize=None)
def _make_sc_gather(TOT, GPW):
    CH = GPW * 128

    @functools.partial(
        pl.kernel,
        mesh=plsc.VectorSubcoreMesh(core_axis_name="c", subcore_axis_name="s"),
        compiler_params=pltpu.CompilerParams(use_tc_tiling_on_sc=False),
        out_type=jax.ShapeDtypeStruct((TOT, 8), jnp.float32),
        scratch_types=[
            pltpu.VMEM((GPW, 128), jnp.int32),
            pltpu.VMEM((CH, 8), jnp.float32),
        ],
    )
    def gath(tab_hbm, idx_hbm, out_hbm, idx_v, rows_v):
        c = lax.axis_index("c")
        s = lax.axis_index("s")
        wid = s * _SC_NC + c
        pltpu.sync_copy(idx_hbm.at[wid], idx_v)

        def body(g, _):
            pltpu.sync_copy(tab_hbm.at[idx_v.at[g]],
                            rows_v.at[pl.ds(g * 128, 128)])
            return _

        lax.fori_loop(0, GPW, body, None)
        pltpu.sync_copy(rows_v, out_hbm.at[pl.ds(wid * CH, CH)])

    return gath


def _mono_ref(ps, x):
    t = jnp.array([1.0, -1.0], dtype=x.dtype)
    n = len(ps)
    for idx in range(n):
        W, b = ps[idx]
        Wm = jnp.abs(W) * t if idx == 0 else jnp.abs(W)
        x = x @ Wm.T + b
        if idx < n - 1:
            x = jax.nn.softplus(x)
    return x


def kernel(v, edge_index, r_ij, S, d, dW, dV, params):
    N = v.shape[0]
    E = edge_index.shape[1]
    i = edge_index[0]
    j = edge_index[1]
    kB = jnp.exp(params['log_k_B'])
    m = jnp.exp(params['log_m'])
    invm = 1.0 / m
    s2kB = jnp.sqrt(2 * kB)

    V = 1.0 / d
    S_perturb = jnp.concatenate([S, S + EPS2, S, S - EPS2], axis=0)
    V_perturb = jnp.concatenate([V, V, V + EPS2, V], axis=0)
    U_cat = _mono_ref(params['U'],
                      jnp.concatenate([S_perturb, V_perturb], axis=-1))
    U = U_cat[0:N]
    U_Splus = U_cat[N:2 * N]
    U_Vplus = U_cat[2 * N:3 * N]
    U_Sminus = U_cat[3 * N:4 * N]
    T = (U_Splus - U) / EPS2
    P = -(U_Vplus - U) / EPS2
    C = T * EPS2 ** 2 / (U_Splus - 2 * U + U_Sminus)
    invT = 1.0 / T
    p2 = P / d ** 2
    invC = 1.0 / C
    invCT = 1.0 / C / T

    F = jnp.concatenate([v, T, invT, p2, invC, invCT], axis=1)
    E2 = 2 * E
    GPW = -(-E2 // (_SC_NW * 128))
    TOT = _SC_NW * GPW * 128
    ij_g = jnp.concatenate(
        [i, j, jnp.zeros((TOT - E2,), jnp.int32)]).reshape(_SC_NW, GPW, 128)
    g_rows = _make_sc_gather(TOT, GPW)(F, ij_g)

    rT = r_ij.T
    dWT = dW.reshape(E, 9).T
    dVT = dV.T

    (W1A, b1A), (W2A, b2A), (W3A, b3A) = params['A']
    (W1B, b1B), (W2B, b2B), (W3B, b3B) = params['B']
    (W1C, b1C), (W2C, b2C), (W3C, b3C) = params['C']
    (W1W, b1W), (W2W, b2W), (W3W, b3W) = params['W']
    z32 = jnp.zeros((32,), jnp.float32)
    P1 = jnp.stack([W1A[:, 0], W1A[:, 1], W1B[:, 0], W1B[:, 1],
                    W1C[:, 0], W1C[:, 1], W1W[:, 0], z32], axis=1)
    B12 = jnp.stack([b1A, b1B, b1C, b1W, b2A, b2B, b2C, b2W], axis=1)
    W2s = jnp.concatenate([W2A, W2B, W2C, W2W], axis=1)
    W3s = jnp.concatenate([W3A, W3B, W3C, W3W], axis=0)
    SCL = jnp.stack([b3A[0], b3B[0], b3C[0], b3W[0],
                     kB, invm, s2kB * invm,
                     jnp.float32(0.0)]).reshape(1, 8)

    nblk = E // BE
    EP = ((E + 5119) // 5120) * 5120
    full = lambda n: (0, n)
    rows = lambda n: (n, 0)
    fix = lambda n: (0, 0)
    pi_rows, pj_rows = pl.pallas_call(
        _edge_body,
        grid=(nblk,),
        in_specs=[
            pl.BlockSpec((3, BE), full),
            pl.BlockSpec((1, BE), full),
            pl.BlockSpec((9, BE), full),
            pl.BlockSpec((BE, 8), rows),
            pl.BlockSpec((BE, 8), lambda n: (nblk + n, 0)),
            pl.BlockSpec((32, 8), fix),
            pl.BlockSpec((32, 8), fix),
            pl.BlockSpec((32, 128), fix),
            pl.BlockSpec((4, 32), fix),
            pl.BlockSpec((1, 8), fix),
        ],
        out_specs=[pl.BlockSpec((BE, 8), rows),
                   pl.BlockSpec((BE, 8), rows)],
        out_shape=[jax.ShapeDtypeStruct((EP, 8), jnp.float32),
                   jax.ShapeDtypeStruct((EP, 8), jnp.float32)],
    )(rT, dVT, dWT, g_rows, g_rows, P1, B12, W2s, W3s, SCL)

    NACC = ((N + 1 + 127) // 128) * 128
    GW = EP // 128 // _SC_NW
    pad = jnp.full((EP - E,), N, jnp.int32)
    ij = jnp.stack([jnp.concatenate([i, pad]),
                    jnp.concatenate([j, pad])]).reshape(2, EP // 128, 128)
    zeros_acc = jnp.zeros((NACC, 8), jnp.float32)
    partials = _make_sc_scatter(EP, GW, NACC)(pi_rows, pj_rows, ij, zeros_acc)
    acc = partials[0, :N] + partials[1, :N]
    out = jnp.concatenate([acc[:, 0:3],
                           acc[:, 3:4] * invT,
                           acc[:, 4:7],
                           acc[:, 7:8] * (s2kB * invT)], axis=1)
    return out


# speedup vs baseline: 16.6959x; 1.1369x over previous
import functools

import jax
import jax.numpy as jnp
from jax import lax
from jax.experimental import pallas as pl
from jax.experimental.pallas import tpu as pltpu
from jax.experimental.pallas import tpu_sc as plsc

H_SMOOTH = 2.0
EPS = 0.001
EPS2 = 0.01
EPS3 = 0.001
BE = 6400

def _dot1(a, b):
    return lax.dot_general(a, b, (((1,), (0,)), ((), ())),
                           preferred_element_type=jnp.float32)


def _dotg(a, b):
    ah = a.astype(jnp.bfloat16)
    al = (a - ah.astype(jnp.float32)).astype(jnp.bfloat16)
    bh = b.astype(jnp.bfloat16)
    bl = (b - bh.astype(jnp.float32)).astype(jnp.bfloat16)
    return _dot1(ah, bh) + _dot1(ah, bl) + _dot1(al, bh)


def _edge_body(rT, dVT, dWT, fi_blk, fj_blk, P1, B12, W2s, W3s, SCL,
               pi_ref, pj_ref):
    fiT = jnp.transpose(fi_blk[...], (1, 0))
    fjT = jnp.transpose(fj_blk[...], (1, 0))
    rij = rT[...]
    r = jnp.sqrt(jnp.sum(rij * rij, axis=0, keepdims=True))
    e = rij / (r + 1e-8)
    s = r / H_SMOOTH

    vi = fiT[0:3, :]
    vj = fjT[0:3, :]
    Ti = fiT[3:4, :]
    Tj = fjT[3:4, :]
    invTi = fiT[4:5, :]
    invTj = fjT[4:5, :]
    p2i = fiT[5:6, :]
    p2j = fjT[5:6, :]
    invCi = fiT[6:7, :]
    invCj = fjT[6:7, :]
    invCTi = fiT[7:8, :]
    invCTj = fjT[7:8, :]

    vij = vi - vj
    ev = jnp.sum(e * vij, axis=0, keepdims=True)
    vv = jnp.sum(vij * vij, axis=0, keepdims=True)

    kB = SCL[0:1, 4:5]
    invm = SCL[0:1, 5:6]
    s2kB_invm = SCL[0:1, 6:7]

    def mlp_abc(c, Trow):
        ws = P1[:, 2 * c:2 * c + 1]
        wt = P1[:, 2 * c + 1:2 * c + 2]
        b1 = B12[:, c:c + 1]
        b2 = B12[:, 4 + c:4 + c + 1]
        h1 = jax.nn.silu(ws * s + wt * Trow + b1)
        h2 = jax.nn.silu(_dotg(W2s[:, 32 * c:32 * c + 32], h1) + b2)
        return _dotg(W3s[c:c + 1, :], h2) + SCL[0:1, c:c + 1]

    def mlp_w(sp):
        h1 = jax.nn.silu(P1[:, 6:7] * sp + B12[:, 3:4])
        h2 = jax.nn.silu(_dotg(W2s[:, 96:128], h1) + B12[:, 7:8])
        out = _dotg(W3s[3:4, :], h2) + SCL[0:1, 3:4]
        return jnp.exp(out) * (1.0 - sp * sp)

    sp_p = (r + EPS) / H_SMOOTH
    sp_m = (r - EPS) / H_SMOOTH
    dW_dr = (mlp_w(sp_p) - mlp_w(sp_m)) / (2 * EPS)
    gW = dW_dr * e

    Ai = mlp_abc(0, Ti)
    Aj = mlp_abc(0, Tj)
    Aie = mlp_abc(0, Ti + EPS3)
    Aje = mlp_abc(0, Tj + EPS3)
    Bi = mlp_abc(1, Ti)
    Bj = mlp_abc(1, Tj)
    Bie = mlp_abc(1, Ti + EPS3)
    Bje = mlp_abc(1, Tj + EPS3)
    Ci = mlp_abc(2, Ti)
    Cj = mlp_abc(2, Tj)
    Cie = mlp_abc(2, Ti + EPS3)
    Cje = mlp_abc(2, Tj + EPS3)

    Aij = Ai * Aj
    Bij = Bi * Bj
    Cij = Ci * Cj
    gA_i = 2.0 * Aij * (Aie * Aj - Aij) / EPS3
    gB_i = 2.0 * Bij * (Bie * Bj - Bij) / EPS3
    gC_i = 2.0 * Cij * (Cie * Cj - Cij) / EPS3
    gA_j = 2.0 * Aij * (Ai * Aje - Aij) / EPS3
    gB_j = 2.0 * Bij * (Bi * Bje - Bij) / EPS3
    gC_j = 2.0 * Cij * (Ci * Cje - Cij) / EPS3

    A2 = Aij * Aij
    B2 = Bij * Bij
    C2 = Cij * Cij

    termPd = (p2i + p2j) * gW
    aux = A2 / 2 * vij + (A2 / 2 + (B2 - A2) / 3.0) * ev * e
    term = (invTi + invTj) * aux
    term1 = -(invCTi + invCTj) * aux
    term2 = ((gA_i / 2 * vij + (gA_i / 2 + (gB_i - gA_i) / 3.0) * ev * e)
             * invCi
             + (gA_j / 2 * vij + (gA_j / 2 + (gB_j - gA_j) / 3.0) * ev * e)
             * invCj)
    msg_v = termPd + 0.5 * term + 0.5 * kB * (term1 + term2)
    pvec_i = -invm * msg_v
    pvec_j = invm * msg_v

    aux2 = (A2 / 2 * vv + (A2 / 2 + (B2 - A2) / 3.0) * ev * ev) / 4
    t1 = -(2 * invCTi + invCTj) * aux2
    t2 = ((gA_i / 2 * vv + (gA_i / 2 + (gB_i - gA_i) / 3.0) * ev * ev)
          * invCi / 4
          + (gA_j / 2 * vv + (gA_j / 2 + (gB_j - gA_j) / 3.0) * ev * ev)
          * invCj / 4)
    t4 = -(2 * invCTi - invCTj) * C2
    t5 = gC_i * invCi - gC_j * invCj
    t6 = -(4.0 * A2 / 2 + (B2 - A2) / 3.0)
    common = (invTi + invTj) * aux2 + kB * (t1 + t2 + t6 * invm)
    anti = (invTi - invTj) * C2 + kB * (t4 + t5)
    ps_i = common + anti
    ps_j = common - anti

    dw = dWT[...]
    tr3 = (dw[0:1] + dw[4:5] + dw[8:9]) / 3.0
    q01 = (dw[1:2] + dw[3:4]) * 0.5
    q02 = (dw[2:3] + dw[6:7]) * 0.5
    q12 = (dw[5:6] + dw[7:8]) * 0.5
    e0 = e[0:1]
    e1 = e[1:2]
    e2 = e[2:3]
    sde0 = dw[0:1] * e0 + q01 * e1 + q02 * e2
    sde1 = q01 * e0 + dw[4:5] * e1 + q12 * e2
    sde2 = q02 * e0 + q12 * e1 + dw[8:9] * e2
    symdote = jnp.concatenate([sde0, sde1, sde2], axis=0)
    wterm = Aij * symdote + (Bij - Aij) * tr3 * e
    pw_i = s2kB_invm * wterm
    pw_j = -pw_i

    st = -0.5 * jnp.sum(wterm * vij, axis=0, keepdims=True)
    sc = Cij * dVT[...]
    pt_i = st + sc
    pt_j = st - sc

    pi_ref[...] = jnp.transpose(
        jnp.concatenate([pvec_i, ps_i, pw_i, pt_i], axis=0), (1, 0))
    pj_ref[...] = jnp.transpose(
        jnp.concatenate([pvec_j, ps_j, pw_j, pt_j], axis=0), (1, 0))


_SC_NC = 2
_SC_NS = 16
_SC_NW = _SC_NC * _SC_NS


@functools.lru_cache(maxsize=None)
def _make_sc_scatter(EP, GW, NACC):
    CH = GW * 128
    SUBROWS = NACC // _SC_NS
    mesh = plsc.VectorSubcoreMesh(core_axis_name="c", subcore_axis_name="s")

    @functools.partial(
        pl.kernel, mesh=mesh,
        compiler_params=pltpu.CompilerParams(use_tc_tiling_on_sc=False),
        out_type=jax.ShapeDtypeStruct((_SC_NC, NACC, 8), jnp.float32),
        scratch_types=[
            pltpu.VMEM((GW, 128), jnp.int32),
            pltpu.VMEM((CH, 8), jnp.float32),
            pltpu.VMEM_SHARED((NACC, 8), jnp.float32),
        ],
    )
    def scat(pi_hbm, pj_hbm, idx_hbm, zeros_hbm, out_hbm,
             idx_v, pay_v, acc_sh):
        c = lax.axis_index("c")
        s = lax.axis_index("s")
        wid = s * _SC_NC + c
        pltpu.sync_copy(zeros_hbm.at[pl.ds(s * SUBROWS, SUBROWS)],
                        acc_sh.at[pl.ds(s * SUBROWS, SUBROWS)])
        plsc.subcore_barrier()

        def one_stream(t, pay_hbm):
            pltpu.sync_copy(idx_hbm.at[t, pl.ds(wid * GW, GW)], idx_v)
            pltpu.sync_copy(pay_hbm.at[pl.ds(wid * CH, CH)], pay_v)

            def body(g, _):
                pltpu.sync_copy(pay_v.at[pl.ds(g * 128, 128)],
                                acc_sh.at[idx_v.at[g]], add=True)
                return _

            lax.fori_loop(0, GW, body, None)

        one_stream(0, pi_hbm)
        one_stream(1, pj_hbm)
        plsc.subcore_barrier()
        pltpu.sync_copy(acc_sh.at[pl.ds(s * SUBROWS, SUBROWS)],
                        out_hbm.at[c, pl.ds(s * SUBROWS, SUBROWS)])

    return scat


@functools.lru_cache(maxsize=None)
def _make_sc_gather(TOT, GPW):
    CH = GPW * 128

    @functools.partial(
        pl.kernel,
        mesh=plsc.VectorSubcoreMesh(core_axis_name="c", subcore_axis_name="s"),
        compiler_params=pltpu.CompilerParams(use_tc_tiling_on_sc=False),
        out_type=jax.ShapeDtypeStruct((TOT, 8), jnp.float32),
        scratch_types=[
            pltpu.VMEM((GPW, 128), jnp.int32),
            pltpu.VMEM((CH, 8), jnp.float32),
        ],
    )
    def gath(tab_hbm, idx_hbm, out_hbm, idx_v, rows_v):
        c = lax.axis_index("c")
        s = lax.axis_index("s")
        wid = s * _SC_NC + c
        pltpu.sync_copy(idx_hbm.at[wid], idx_v)

        def body(g, _):
            pltpu.sync_copy(tab_hbm.at[idx_v.at[g]],
                            rows_v.at[pl.ds(g * 128, 128)])
            return _

        lax.fori_loop(0, GPW, body, None)
        pltpu.sync_copy(rows_v, out_hbm.at[pl.ds(wid * CH, CH)])

    return gath


def _mono_ref(ps, x):
    t = jnp.array([1.0, -1.0], dtype=x.dtype)
    n = len(ps)
    for idx in range(n):
        W, b = ps[idx]
        Wm = jnp.abs(W) * t if idx == 0 else jnp.abs(W)
        x = x @ Wm.T + b
        if idx < n - 1:
            x = jax.nn.softplus(x)
    return x


def kernel(v, edge_index, r_ij, S, d, dW, dV, params):
    N = v.shape[0]
    E = edge_index.shape[1]
    i = edge_index[0]
    j = edge_index[1]
    kB = jnp.exp(params['log_k_B'])
    m = jnp.exp(params['log_m'])
    invm = 1.0 / m
    s2kB = jnp.sqrt(2 * kB)

    V = 1.0 / d
    S_perturb = jnp.concatenate([S, S + EPS2, S, S - EPS2], axis=0)
    V_perturb = jnp.concatenate([V, V, V + EPS2, V], axis=0)
    U_cat = _mono_ref(params['U'],
                      jnp.concatenate([S_perturb, V_perturb], axis=-1))
    U = U_cat[0:N]
    U_Splus = U_cat[N:2 * N]
    U_Vplus = U_cat[2 * N:3 * N]
    U_Sminus = U_cat[3 * N:4 * N]
    T = (U_Splus - U) / EPS2
    P = -(U_Vplus - U) / EPS2
    C = T * EPS2 ** 2 / (U_Splus - 2 * U + U_Sminus)
    invT = 1.0 / T
    p2 = P / d ** 2
    invC = 1.0 / C
    invCT = 1.0 / C / T

    F = jnp.concatenate([v, T, invT, p2, invC, invCT], axis=1)
    E2 = 2 * E
    GPW = -(-E2 // (_SC_NW * 128))
    TOT = _SC_NW * GPW * 128
    ij_g = jnp.concatenate(
        [i, j, jnp.zeros((TOT - E2,), jnp.int32)]).reshape(_SC_NW, GPW, 128)
    g_rows = _make_sc_gather(TOT, GPW)(F, ij_g)

    rT = r_ij.T
    dWT = dW.reshape(E, 9).T
    dVT = dV.T

    (W1A, b1A), (W2A, b2A), (W3A, b3A) = params['A']
    (W1B, b1B), (W2B, b2B), (W3B, b3B) = params['B']
    (W1C, b1C), (W2C, b2C), (W3C, b3C) = params['C']
    (W1W, b1W), (W2W, b2W), (W3W, b3W) = params['W']
    z32 = jnp.zeros((32,), jnp.float32)
    P1 = jnp.stack([W1A[:, 0], W1A[:, 1], W1B[:, 0], W1B[:, 1],
                    W1C[:, 0], W1C[:, 1], W1W[:, 0], z32], axis=1)
    B12 = jnp.stack([b1A, b1B, b1C, b1W, b2A, b2B, b2C, b2W], axis=1)
    W2s = jnp.concatenate([W2A, W2B, W2C, W2W], axis=1)
    W3s = jnp.concatenate([W3A, W3B, W3C, W3W], axis=0)
    SCL = jnp.stack([b3A[0], b3B[0], b3C[0], b3W[0],
                     kB, invm, s2kB * invm,
                     jnp.float32(0.0)]).reshape(1, 8)

    nblk = E // BE
    EP = ((E + 5119) // 5120) * 5120
    full = lambda n: (0, n)
    rows = lambda n: (n, 0)
    fix = lambda n: (0, 0)
    pi_rows, pj_rows = pl.pallas_call(
        _edge_body,
        grid=(nblk,),
        in_specs=[
            pl.BlockSpec((3, BE), full),
            pl.BlockSpec((1, BE), full),
            pl.BlockSpec((9, BE), full),
            pl.BlockSpec((BE, 8), rows),
            pl.BlockSpec((BE, 8), lambda n: (nblk + n, 0)),
            pl.BlockSpec((32, 8), fix),
            pl.BlockSpec((32, 8), fix),
            pl.BlockSpec((32, 128), fix),
            pl.BlockSpec((4, 32), fix),
            pl.BlockSpec((1, 8), fix),
        ],
        out_specs=[pl.BlockSpec((BE, 8), rows),
                   pl.BlockSpec((BE, 8), rows)],
        out_shape=[jax.ShapeDtypeStruct((EP, 8), jnp.float32),
                   jax.ShapeDtypeStruct((EP, 8), jnp.float32)],
    )(rT, dVT, dWT, g_rows, g_rows, P1, B12, W2s, W3s, SCL)

    NACC = ((N + 1 + 127) // 128) * 128
    GW = EP // 128 // _SC_NW
    pad = jnp.full((EP - E,), N, jnp.int32)
    ij = jnp.stack([jnp.concatenate([i, pad]),
                    jnp.concatenate([j, pad])]).reshape(2, EP // 128, 128)
    zeros_acc = jnp.zeros((NACC, 8), jnp.float32)
    partials = _make_sc_scatter(EP, GW, NACC)(pi_rows, pj_rows, ij, zeros_acc)
    acc = partials[0, :N] + partials[1, :N]
    out = jnp.concatenate([acc[:, 0:3],
                           acc[:, 3:4] * invT,
                           acc[:, 4:7],
                           acc[:, 7:8] * (s2kB * invT)], axis=1)
    return out
